# baseline (speedup 1.0000x reference)
import functools

import jax
import jax.numpy as jnp
from jax import lax
from jax.experimental import pallas as pl
from jax.experimental.pallas import tpu as pltpu
from jax.experimental.pallas import tpu_sc as plsc

_ROWS, _COLS = 16384, 4096
_N = _ROWS * _COLS
_NC, _NS, _L = 2, 16, 16
_NW = _NC * _NS
_PER_W = _N // _NW
_CH = 32768
_NCH = _PER_W // _CH
_UNROLL = 8


def _make_sc_kernel():
    mesh = plsc.VectorSubcoreMesh(core_axis_name="c", subcore_axis_name="s")

    @functools.partial(
        pl.kernel,
        mesh=mesh,
        out_type=jax.ShapeDtypeStruct((_N,), jnp.float32),
        scratch_types=[
            pltpu.VMEM((2, _CH), jnp.float32),
            pltpu.SemaphoreType.DMA((2,)),
            pltpu.SemaphoreType.DMA((2,)),
        ],
    )
    def k(in_hbm, out_hbm, buf, in_sem, out_sem):
        wid = lax.axis_index("s") * _NC + lax.axis_index("c")
        base = wid * _PER_W

        def start_in(j, slot):
            pltpu.make_async_copy(
                in_hbm.at[pl.ds(base + j * _CH, _CH)], buf.at[slot],
                in_sem.at[slot],
            ).start()

        def wait_in(j, slot):
            pltpu.make_async_copy(
                in_hbm.at[pl.ds(base + j * _CH, _CH)], buf.at[slot],
                in_sem.at[slot],
            ).wait()

        def start_out(j, slot):
            pltpu.make_async_copy(
                buf.at[slot], out_hbm.at[pl.ds(base + j * _CH, _CH)],
                out_sem.at[slot],
            ).start()

        def wait_out(j, slot):
            pltpu.make_async_copy(
                buf.at[slot], out_hbm.at[pl.ds(base + j * _CH, _CH)],
                out_sem.at[slot],
            ).wait()

        start_in(0, 0)

        ones = jnp.full((_L,), 1.0, jnp.float32)
        zeros = jnp.zeros((_L,), jnp.float32)

        def process(j, slot, nslot):
            wait_in(j, slot)

            @pl.when(j + 1 < _NCH)
            def _prefetch():
                @pl.when(j >= 1)
                def _():
                    wait_out(j - 1, nslot)
                start_in(j + 1, nslot)

            def vec_body(i, c2):
                b = i * (_L * _UNROLL)
                for u in range(_UNROLL):
                    off = b + u * _L
                    v = buf[slot, pl.ds(off, _L)]
                    buf[slot, pl.ds(off, _L)] = jnp.where(v != 0.0, ones, zeros)
                return c2

            lax.fori_loop(0, _CH // (_L * _UNROLL), vec_body, 0)
            start_out(j, slot)

        def body(jj, carry):
            j = jj * 2
            process(j, 0, 1)
            process(j + 1, 1, 0)
            return carry

        lax.fori_loop(0, _NCH // 2, body, 0)
        wait_out(_NCH - 2, lax.rem(_NCH - 2, 2))
        wait_out(_NCH - 1, lax.rem(_NCH - 1, 2))

    return k


_sc_kernel = _make_sc_kernel()


def kernel(mask):
    flat = mask.reshape(_N)
    return _sc_kernel(flat).reshape(_ROWS, _COLS)
